# baseline (speedup 1.0000x reference)
import functools
import math

import jax
import jax.numpy as jnp
from jax import lax
from jax.experimental import pallas as pl
from jax.experimental.pallas import tpu as pltpu
from jax.experimental.pallas import tpu_sc as plsc

D_EMBED = 64
SCALE = math.sqrt(D_EMBED)
L = 16
C = 128
NBUF = 4


def _build_sc_kernel(num_rows_x, num_cols_x):
    try:
        info = plsc.get_sparse_core_info()
        nc, ns = info.num_cores, info.num_subcores
    except Exception:
        nc, ns = 2, 16
    nw = nc * ns
    b_total = num_rows_x * num_cols_x
    assert b_total % (nw * C) == 0
    per_w = b_total // nw
    nchunk = per_w // C
    assert nchunk % NBUF == 0 and nchunk >= 2 * NBUF

    mesh = plsc.VectorSubcoreMesh(core_axis_name="c", subcore_axis_name="s")

    @functools.partial(
        pl.kernel,
        mesh=mesh,
        compiler_params=pltpu.CompilerParams(use_tc_tiling_on_sc=False),
        out_type=jax.ShapeDtypeStruct((b_total, D_EMBED), jnp.float32),
        scratch_types=(
            [pltpu.VMEM((nchunk, C), jnp.int32)]
            + [pltpu.VMEM((C, D_EMBED), jnp.float32) for _ in range(2 * NBUF)]
            + [pltpu.SemaphoreType.DMA for _ in range(2 * NBUF)]
        ),
    )
    def emb(x_hbm, table_hbm, out_hbm, idx_v, *bufs_and_sems):
        gbuf = bufs_and_sems[0:NBUF]
        sbuf = bufs_and_sems[NBUF:2 * NBUF]
        gsem = bufs_and_sems[2 * NBUF:3 * NBUF]
        ssem = bufs_and_sems[3 * NBUF:4 * NBUF]

        wid = lax.axis_index("s") * nc + lax.axis_index("c")
        base = wid * per_w

        pltpu.sync_copy(x_hbm.at[wid], idx_v)

        def start_gather(j, b):
            pltpu.async_copy(table_hbm.at[idx_v.at[j]], gbuf[b], gsem[b])

        def wait_gather(b):
            pltpu.make_async_copy(
                table_hbm.at[idx_v.at[0]], gbuf[b], gsem[b]).wait()

        def start_scatter(j, b):
            pltpu.async_copy(
                sbuf[b], out_hbm.at[pl.ds(base + j * C, C)], ssem[b])

        def wait_scatter(b):
            pltpu.make_async_copy(
                sbuf[b], out_hbm.at[pl.ds(base, C)], ssem[b]).wait()

        def scale(b):
            gb, sb = gbuf[b], sbuf[b]

            @plsc.parallel_loop(0, C, unroll=2)
            def _(r):
                for c4 in range(D_EMBED // L):
                    sl = pl.ds(c4 * L, L)
                    sb[r, sl] = gb[r, sl] * SCALE

        for b in range(NBUF):
            start_gather(b, b)

        for b in range(NBUF):
            wait_gather(b)
            scale(b)
            start_scatter(b, b)
            start_gather(b + NBUF, b)

        @pl.loop(NBUF, nchunk - NBUF, step=NBUF)
        def _(g):
            for b in range(NBUF):
                j = g + b
                wait_gather(b)
                wait_scatter(b)
                scale(b)
                start_scatter(j, b)
                start_gather(j + NBUF, b)

        for b in range(NBUF):
            j = nchunk - NBUF + b
            wait_gather(b)
            wait_scatter(b)
            scale(b)
            start_scatter(j, b)

        for b in range(NBUF):
            wait_scatter(b)

    return emb, nw, nchunk


def kernel(x, table):
    rows, cols = x.shape
    emb, nw, nchunk = _build_sc_kernel(rows, cols)
    xf = x.reshape(nw, nchunk, C).astype(jnp.int32)
    out = emb(xf, table)
    return out.reshape(rows, cols, D_EMBED)
